# Initial kernel scaffold; baseline (speedup 1.0000x reference)
import jax
import jax.numpy as jnp
from jax.experimental import pallas as pl


def kernel(x, denominator_table, numerator_table):
    raise NotImplementedError("write your pallas kernel here")


# speedup vs baseline: 514.5499x; 514.5499x over previous
import functools

import jax
import jax.numpy as jnp
from jax import lax
from jax.experimental import pallas as pl
from jax.experimental.pallas import tpu as pltpu
from jax.experimental.pallas import tpu_sc as plsc

ROWS = 2048
COLS = 32768
TABLE = 256
LANES = 16
NWORKERS = 32
W = COLS // NWORKERS
RC = 32
NCHUNKS = ROWS // RC
NUDGE = float(1.0 + 2.0 ** -18)


def _sc_body(x_hbm, den_hbm, num_hbm, out_hbm,
             xbuf, obuf, denbuf, numbuf, dsbuf, recipbuf):
    info = plsc.get_sparse_core_info()
    nc = info.num_cores
    wid = lax.axis_index("s") * nc + lax.axis_index("c")
    col0 = wid * W
    idx4 = lax.iota(jnp.int32, LANES) * 4

    pltpu.sync_copy(den_hbm, denbuf)
    pltpu.sync_copy(num_hbm, numbuf)

    def zero_cg(cg, c):
        dsbuf[pl.ds(cg * LANES, LANES)] = jnp.zeros((LANES,), jnp.float32)
        return c
    lax.fori_loop(0, W // LANES, zero_cg, 0)

    def p1_chunk(ch, c):
        pltpu.sync_copy(x_hbm.at[pl.ds(ch * RC, RC), pl.ds(col0, W)], xbuf)

        def p1_cg(cg, cc):
            def p1_row(r, acc):
                idx = xbuf[r, pl.ds(cg * LANES, LANES)]
                return acc + plsc.load_gather(denbuf, [idx])
            acc = lax.fori_loop(0, RC, p1_row, jnp.zeros((LANES,), jnp.float32))
            cur = dsbuf[pl.ds(cg * LANES, LANES)]
            dsbuf[pl.ds(cg * LANES, LANES)] = cur + acc
            return cc
        lax.fori_loop(0, W // LANES, p1_cg, 0)
        return c
    lax.fori_loop(0, NCHUNKS, p1_chunk, 0)

    def rec_cg(cg, c):
        ds = dsbuf[pl.ds(cg * LANES, LANES)]
        recipbuf[pl.ds(cg * LANES, LANES)] = jnp.float32(NUDGE) / ds
        return c
    lax.fori_loop(0, W // LANES, rec_cg, 0)

    def p2_chunk(ch, c):
        pltpu.sync_copy(x_hbm.at[pl.ds(ch * RC, RC), pl.ds(col0, W)], xbuf)

        def p2_cg(cg, cc):
            c0 = cg * 64
            recs = [plsc.load_gather(recipbuf, [idx4 + (c0 + j)])
                    for j in range(4)]

            def p2_row(r, rr):
                rvec = jnp.full((LANES,), r, jnp.int32)
                packed = jnp.zeros((LANES,), jnp.int32)
                for j in range(4):
                    ci = idx4 + (c0 + j)
                    xv = plsc.load_gather(xbuf, [rvec, ci])
                    nv = plsc.load_gather(numbuf, [xv])
                    yi = jnp.minimum((nv * recs[j]).astype(jnp.int32), 255)
                    packed = packed | (yi << (8 * j))
                obuf[r, pl.ds(c0, 64)] = plsc.bitcast(packed, jnp.uint8)
                return rr
            lax.fori_loop(0, RC, p2_row, 0)
            return cc
        lax.fori_loop(0, W // 64, p2_cg, 0)
        pltpu.sync_copy(obuf, out_hbm.at[pl.ds(ch * RC, RC), pl.ds(col0, W)])
        return c
    lax.fori_loop(0, NCHUNKS, p2_chunk, 0)


_softmax_sc = functools.partial(
    pl.kernel,
    out_type=jax.ShapeDtypeStruct((ROWS, COLS), jnp.uint8),
    mesh=plsc.VectorSubcoreMesh(core_axis_name="c", subcore_axis_name="s"),
    compiler_params=pltpu.CompilerParams(needs_layout_passes=False,
                                         use_tc_tiling_on_sc=False),
    scratch_types=[
        pltpu.VMEM((RC, W), jnp.int32),
        pltpu.VMEM((RC, W), jnp.uint8),
        pltpu.VMEM((TABLE,), jnp.float32),
        pltpu.VMEM((TABLE,), jnp.float32),
        pltpu.VMEM((W,), jnp.float32),
        pltpu.VMEM((W,), jnp.float32),
    ],
)(_sc_body)


def kernel(x, denominator_table, numerator_table):
    return _softmax_sc(x, denominator_table, numerator_table)


# speedup vs baseline: 799.0122x; 1.5528x over previous
import functools

import jax
import jax.numpy as jnp
from jax import lax
from jax.experimental import pallas as pl
from jax.experimental.pallas import tpu as pltpu
from jax.experimental.pallas import tpu_sc as plsc

ROWS = 2048
COLS = 32768
TABLE = 256
LANES = 16
NWORKERS = 32
W = COLS // NWORKERS
RC = 32
NCHUNKS = ROWS // RC
NUDGE = float(1.0 + 2.0 ** -18)


def _sc_body(x_hbm, den_hbm, num_hbm, out_hbm,
             xbuf0, xbuf1, obuf0, obuf1, denbuf, numbuf, dsbuf, recipbuf,
             semx0, semx1, semo0, semo1):
    info = plsc.get_sparse_core_info()
    nc = info.num_cores
    wid = lax.axis_index("s") * nc + lax.axis_index("c")
    col0 = wid * W
    idx4 = lax.iota(jnp.int32, LANES) * 4
    xb = (xbuf0, xbuf1)
    ob = (obuf0, obuf1)
    sx = (semx0, semx1)
    so = (semo0, semo1)

    def xslice(ch):
        return x_hbm.at[pl.ds(ch * RC, RC), pl.ds(col0, W)]

    def oslice(ch):
        return out_hbm.at[pl.ds(ch * RC, RC), pl.ds(col0, W)]

    pltpu.sync_copy(den_hbm, denbuf)
    pltpu.sync_copy(num_hbm, numbuf)

    def zero_cg(cg, c):
        dsbuf[pl.ds(cg * LANES, LANES)] = jnp.zeros((LANES,), jnp.float32)
        return c
    lax.fori_loop(0, W // LANES, zero_cg, 0)

    pltpu.async_copy(xslice(0), xb[0], sx[0])

    def p1_pair(p, c):
        for b in range(2):
            ch = p * 2 + b

            @pl.when(ch + 1 < NCHUNKS)
            def _():
                pltpu.async_copy(xslice(ch + 1), xb[1 - b], sx[1 - b])

            pltpu.make_async_copy(xslice(ch), xb[b], sx[b]).wait()
            xbuf = xb[b]

            def p1_cg(cg, cc):
                def p1_row8(r8, acc):
                    r = r8 * 8
                    for k in range(8):
                        idx = xbuf[r + k, pl.ds(cg * LANES, LANES)]
                        acc = acc + plsc.load_gather(denbuf, [idx])
                    return acc
                acc = lax.fori_loop(0, RC // 8, p1_row8,
                                    jnp.zeros((LANES,), jnp.float32))
                cur = dsbuf[pl.ds(cg * LANES, LANES)]
                dsbuf[pl.ds(cg * LANES, LANES)] = cur + acc
                return cc
            lax.fori_loop(0, W // LANES, p1_cg, 0)
        return c
    lax.fori_loop(0, NCHUNKS // 2, p1_pair, 0)

    def rec_cg(cg, c):
        ds = dsbuf[pl.ds(cg * LANES, LANES)]
        recipbuf[pl.ds(cg * LANES, LANES)] = jnp.float32(NUDGE) / ds
        return c
    lax.fori_loop(0, W // LANES, rec_cg, 0)

    pltpu.async_copy(xslice(0), xb[0], sx[0])

    def p2_pair(p, c):
        for b in range(2):
            ch = p * 2 + b

            @pl.when(ch + 1 < NCHUNKS)
            def _():
                pltpu.async_copy(xslice(ch + 1), xb[1 - b], sx[1 - b])

            pltpu.make_async_copy(xslice(ch), xb[b], sx[b]).wait()

            @pl.when(ch >= 2)
            def _():
                pltpu.make_async_copy(ob[b], oslice(ch - 2), so[b]).wait()

            xbuf = xb[b]
            obuf = ob[b]

            def p2_cg(cg, cc):
                c0 = cg * 64
                cis = [idx4 + (c0 + j) for j in range(4)]
                recs = [plsc.load_gather(recipbuf, [cis[j]]) for j in range(4)]

                def p2_row2(r2, rr):
                    for k in range(2):
                        r = r2 * 2 + k
                        rvec = jnp.full((LANES,), r, jnp.int32)
                        packed = jnp.zeros((LANES,), jnp.int32)
                        for j in range(4):
                            xv = plsc.load_gather(xbuf, [rvec, cis[j]])
                            nv = plsc.load_gather(numbuf, [xv])
                            yi = jnp.minimum((nv * recs[j]).astype(jnp.int32),
                                             255)
                            packed = packed | (yi << (8 * j))
                        obuf[r, pl.ds(c0, 64)] = plsc.bitcast(packed,
                                                              jnp.uint8)
                    return rr
                lax.fori_loop(0, RC // 2, p2_row2, 0)
                return cc
            lax.fori_loop(0, W // 64, p2_cg, 0)
            pltpu.async_copy(obuf, oslice(ch), so[b])
        return c
    lax.fori_loop(0, NCHUNKS // 2, p2_pair, 0)

    pltpu.make_async_copy(ob[0], oslice(NCHUNKS - 2), so[0]).wait()
    pltpu.make_async_copy(ob[1], oslice(NCHUNKS - 1), so[1]).wait()


_softmax_sc = functools.partial(
    pl.kernel,
    out_type=jax.ShapeDtypeStruct((ROWS, COLS), jnp.uint8),
    mesh=plsc.VectorSubcoreMesh(core_axis_name="c", subcore_axis_name="s"),
    compiler_params=pltpu.CompilerParams(needs_layout_passes=False,
                                         use_tc_tiling_on_sc=False),
    scratch_types=[
        pltpu.VMEM((RC, W), jnp.int32),
        pltpu.VMEM((RC, W), jnp.int32),
        pltpu.VMEM((RC, W), jnp.uint8),
        pltpu.VMEM((RC, W), jnp.uint8),
        pltpu.VMEM((TABLE,), jnp.float32),
        pltpu.VMEM((TABLE,), jnp.float32),
        pltpu.VMEM((W,), jnp.float32),
        pltpu.VMEM((W,), jnp.float32),
        pltpu.SemaphoreType.DMA,
        pltpu.SemaphoreType.DMA,
        pltpu.SemaphoreType.DMA,
        pltpu.SemaphoreType.DMA,
    ],
)(_sc_body)


def kernel(x, denominator_table, numerator_table):
    return _softmax_sc(x, denominator_table, numerator_table)


# speedup vs baseline: 801.8517x; 1.0036x over previous
import functools

import jax
import jax.numpy as jnp
from jax import lax
from jax.experimental import pallas as pl
from jax.experimental.pallas import tpu as pltpu
from jax.experimental.pallas import tpu_sc as plsc

ROWS = 2048
COLS = 32768
TABLE = 256
LANES = 16
NWORKERS = 32
W = COLS // NWORKERS
RC = 32
NCHUNKS = ROWS // RC
NUDGE = float(1.0 + 2.0 ** -18)


def _sc_body(x_hbm, den_hbm, num_hbm, out_hbm,
             xbuf0, xbuf1, obuf0, obuf1, denbuf, numbuf, dsbuf, recipbuf,
             semx0, semx1, semo0, semo1):
    info = plsc.get_sparse_core_info()
    nc = info.num_cores
    wid = lax.axis_index("s") * nc + lax.axis_index("c")
    col0 = wid * W
    idx4 = lax.iota(jnp.int32, LANES) * 4
    xb = (xbuf0, xbuf1)
    ob = (obuf0, obuf1)
    sx = (semx0, semx1)
    so = (semo0, semo1)

    def xslice(ch):
        return x_hbm.at[pl.ds(ch * RC, RC), pl.ds(col0, W)]

    def oslice(ch):
        return out_hbm.at[pl.ds(ch * RC, RC), pl.ds(col0, W)]

    pltpu.sync_copy(den_hbm, denbuf)
    pltpu.sync_copy(num_hbm, numbuf)

    def zero_cg(cg, c):
        dsbuf[pl.ds(cg * LANES, LANES)] = jnp.zeros((LANES,), jnp.float32)
        return c
    lax.fori_loop(0, W // LANES, zero_cg, 0)

    pltpu.async_copy(xslice(0), xb[0], sx[0])

    def p1_pair(p, c):
        for b in range(2):
            ch = p * 2 + b

            @pl.when(ch + 1 < NCHUNKS)
            def _():
                pltpu.async_copy(xslice(ch + 1), xb[1 - b], sx[1 - b])

            pltpu.make_async_copy(xslice(ch), xb[b], sx[b]).wait()
            xbuf = xb[b]

            def p1_cg(cg, cc):
                sl = pl.ds(cg * LANES, LANES)
                accs = [jnp.zeros((LANES,), jnp.float32) for _ in range(4)]
                for r in range(RC):
                    idx = xbuf[r, sl]
                    accs[r % 4] = accs[r % 4] + plsc.load_gather(denbuf, [idx])
                acc = (accs[0] + accs[1]) + (accs[2] + accs[3])
                dsbuf[sl] = dsbuf[sl] + acc
                return cc
            lax.fori_loop(0, W // LANES, p1_cg, 0)
        return c
    lax.fori_loop(0, NCHUNKS // 2, p1_pair, 0)

    def rec_cg(cg, c):
        ds = dsbuf[pl.ds(cg * LANES, LANES)]
        recipbuf[pl.ds(cg * LANES, LANES)] = jnp.float32(NUDGE) / ds
        return c
    lax.fori_loop(0, W // LANES, rec_cg, 0)

    pltpu.async_copy(xslice(0), xb[0], sx[0])

    def p2_pair(p, c):
        for b in range(2):
            ch = p * 2 + b

            @pl.when(ch + 1 < NCHUNKS)
            def _():
                pltpu.async_copy(xslice(ch + 1), xb[1 - b], sx[1 - b])

            pltpu.make_async_copy(xslice(ch), xb[b], sx[b]).wait()

            @pl.when(ch >= 2)
            def _():
                pltpu.make_async_copy(ob[b], oslice(ch - 2), so[b]).wait()

            xbuf = xb[b]
            obuf = ob[b]

            def p2_cg(cg, cc):
                c0 = cg * 64
                cis = [idx4 + (c0 + j) for j in range(4)]
                recs = [plsc.load_gather(recipbuf, [cis[j]]) for j in range(4)]

                def p2_row2(r2, rr):
                    for k in range(4):
                        r = r2 * 4 + k
                        rvec = jnp.full((LANES,), r, jnp.int32)
                        packed = jnp.zeros((LANES,), jnp.int32)
                        for j in range(4):
                            xv = plsc.load_gather(xbuf, [rvec, cis[j]])
                            nv = plsc.load_gather(numbuf, [xv])
                            yi = jnp.minimum((nv * recs[j]).astype(jnp.int32),
                                             255)
                            packed = packed | (yi << (8 * j))
                        obuf[r, pl.ds(c0, 64)] = plsc.bitcast(packed,
                                                              jnp.uint8)
                    return rr
                lax.fori_loop(0, RC // 4, p2_row2, 0)
                return cc
            lax.fori_loop(0, W // 64, p2_cg, 0)
            pltpu.async_copy(obuf, oslice(ch), so[b])
        return c
    lax.fori_loop(0, NCHUNKS // 2, p2_pair, 0)

    pltpu.make_async_copy(ob[0], oslice(NCHUNKS - 2), so[0]).wait()
    pltpu.make_async_copy(ob[1], oslice(NCHUNKS - 1), so[1]).wait()


_softmax_sc = functools.partial(
    pl.kernel,
    out_type=jax.ShapeDtypeStruct((ROWS, COLS), jnp.uint8),
    mesh=plsc.VectorSubcoreMesh(core_axis_name="c", subcore_axis_name="s"),
    compiler_params=pltpu.CompilerParams(needs_layout_passes=False,
                                         use_tc_tiling_on_sc=False),
    scratch_types=[
        pltpu.VMEM((RC, W), jnp.int32),
        pltpu.VMEM((RC, W), jnp.int32),
        pltpu.VMEM((RC, W), jnp.uint8),
        pltpu.VMEM((RC, W), jnp.uint8),
        pltpu.VMEM((TABLE,), jnp.float32),
        pltpu.VMEM((TABLE,), jnp.float32),
        pltpu.VMEM((W,), jnp.float32),
        pltpu.VMEM((W,), jnp.float32),
        pltpu.SemaphoreType.DMA,
        pltpu.SemaphoreType.DMA,
        pltpu.SemaphoreType.DMA,
        pltpu.SemaphoreType.DMA,
    ],
)(_sc_body)


def kernel(x, denominator_table, numerator_table):
    return _softmax_sc(x, denominator_table, numerator_table)


# speedup vs baseline: 914.1654x; 1.1401x over previous
import functools

import jax
import jax.numpy as jnp
from jax import lax
from jax.experimental import pallas as pl
from jax.experimental.pallas import tpu as pltpu
from jax.experimental.pallas import tpu_sc as plsc

ROWS = 2048
COLS = 32768
TABLE = 256
LANES = 16
NWORKERS = 32
W = COLS // NWORKERS
RC = 32
NCHUNKS = ROWS // RC

_MESH = plsc.VectorSubcoreMesh(core_axis_name="c", subcore_axis_name="s")
_CPARAMS = pltpu.CompilerParams(needs_layout_passes=False,
                                use_tc_tiling_on_sc=True)


def _worker_col0():
    info = plsc.get_sparse_core_info()
    wid = lax.axis_index("s") * info.num_cores + lax.axis_index("c")
    return wid * W


def _p1_body(x_hbm, den_hbm, ds_hbm,
             xbuf0, xbuf1, denbuf, dsbuf, semx0, semx1):
    col0 = _worker_col0()
    xb = (xbuf0, xbuf1)
    sx = (semx0, semx1)

    def xslice(ch):
        return x_hbm.at[pl.ds(ch * RC, RC), pl.ds(col0, W)]

    pltpu.sync_copy(den_hbm, denbuf)

    def zero_cg(cg, c):
        dsbuf[pl.ds(cg * LANES, LANES)] = jnp.zeros((LANES,), jnp.float32)
        return c
    lax.fori_loop(0, W // LANES, zero_cg, 0)

    pltpu.async_copy(xslice(0), xb[0], sx[0])

    def p1_pair(p, c):
        for b in range(2):
            ch = p * 2 + b

            @pl.when(ch + 1 < NCHUNKS)
            def _():
                pltpu.async_copy(xslice(ch + 1), xb[1 - b], sx[1 - b])

            pltpu.make_async_copy(xslice(ch), xb[b], sx[b]).wait()
            xbuf = xb[b]

            def p1_cg(cg, cc):
                sl = pl.ds(cg * LANES, LANES)
                accs = [jnp.zeros((LANES,), jnp.float32) for _ in range(4)]
                for r in range(RC):
                    idx = xbuf[r, sl]
                    accs[r % 4] = accs[r % 4] + plsc.load_gather(denbuf, [idx])
                acc = (accs[0] + accs[1]) + (accs[2] + accs[3])
                dsbuf[sl] = dsbuf[sl] + acc
                return cc
            lax.fori_loop(0, W // LANES, p1_cg, 0)
        return c
    lax.fori_loop(0, NCHUNKS // 2, p1_pair, 0)

    pltpu.sync_copy(dsbuf, ds_hbm.at[pl.ds(col0, W)])


def _p2_body(x_hbm, num_hbm, recip_hbm, out_hbm,
             xbuf0, xbuf1, obuf0, obuf1, numbuf, recipbuf,
             semx0, semx1, semo0, semo1):
    col0 = _worker_col0()
    xb = (xbuf0, xbuf1)
    ob = (obuf0, obuf1)
    sx = (semx0, semx1)
    so = (semo0, semo1)

    def xslice(ch):
        return x_hbm.at[pl.ds(ch * RC, RC), pl.ds(col0, W)]

    def oslice(ch):
        return out_hbm.at[pl.ds(ch * (RC // 4), RC // 4), pl.ds(col0, W)]

    pltpu.sync_copy(num_hbm, numbuf)
    pltpu.sync_copy(recip_hbm.at[pl.ds(col0, W)], recipbuf)

    pltpu.async_copy(xslice(0), xb[0], sx[0])

    def p2_pair(p, c):
        for b in range(2):
            ch = p * 2 + b

            @pl.when(ch + 1 < NCHUNKS)
            def _():
                pltpu.async_copy(xslice(ch + 1), xb[1 - b], sx[1 - b])

            pltpu.make_async_copy(xslice(ch), xb[b], sx[b]).wait()

            @pl.when(ch >= 2)
            def _():
                pltpu.make_async_copy(ob[b], oslice(ch - 2), so[b]).wait()

            xbuf = xb[b]
            obuf = ob[b]

            def p2_cg(cg, cc):
                sl = pl.ds(cg * LANES, LANES)
                rec = recipbuf[sl]

                def p2_row4(r4, rr):
                    packed = jnp.zeros((LANES,), jnp.int32)
                    for k in range(4):
                        xv = xbuf[r4 * 4 + k, sl]
                        nv = plsc.load_gather(numbuf, [xv])
                        yi = jnp.minimum((nv * rec).astype(jnp.int32), 255)
                        packed = packed | (yi << (8 * k))
                    obuf[r4, sl] = packed
                    return rr
                lax.fori_loop(0, RC // 4, p2_row4, 0)
                return cc
            lax.fori_loop(0, W // LANES, p2_cg, 0)
            pltpu.async_copy(obuf, oslice(ch), so[b])
        return c
    lax.fori_loop(0, NCHUNKS // 2, p2_pair, 0)

    pltpu.make_async_copy(ob[0], oslice(NCHUNKS - 2), so[0]).wait()
    pltpu.make_async_copy(ob[1], oslice(NCHUNKS - 1), so[1]).wait()


_sc_pass1 = functools.partial(
    pl.kernel,
    out_type=jax.ShapeDtypeStruct((COLS,), jnp.float32),
    mesh=_MESH, compiler_params=_CPARAMS,
    scratch_types=[
        pltpu.VMEM((RC, W), jnp.int32),
        pltpu.VMEM((RC, W), jnp.int32),
        pltpu.VMEM((TABLE,), jnp.float32),
        pltpu.VMEM((W,), jnp.float32),
        pltpu.SemaphoreType.DMA,
        pltpu.SemaphoreType.DMA,
    ],
)(_p1_body)

_sc_pass2 = functools.partial(
    pl.kernel,
    out_type=jax.ShapeDtypeStruct((ROWS // 4, COLS), jnp.int32),
    mesh=_MESH, compiler_params=_CPARAMS,
    scratch_types=[
        pltpu.VMEM((RC, W), jnp.int32),
        pltpu.VMEM((RC, W), jnp.int32),
        pltpu.VMEM((RC // 4, W), jnp.int32),
        pltpu.VMEM((RC // 4, W), jnp.int32),
        pltpu.VMEM((TABLE,), jnp.float32),
        pltpu.VMEM((W,), jnp.float32),
        pltpu.SemaphoreType.DMA,
        pltpu.SemaphoreType.DMA,
        pltpu.SemaphoreType.DMA,
        pltpu.SemaphoreType.DMA,
    ],
)(_p2_body)


def kernel(x, denominator_table, numerator_table):
    ds = _sc_pass1(x, denominator_table)
    recip = jnp.float32(1.0) / ds
    packed = _sc_pass2(x, numerator_table, recip)
    yb = lax.bitcast_convert_type(packed, jnp.uint8)
    return yb.transpose(0, 2, 1).reshape(ROWS, COLS)


# speedup vs baseline: 945.0172x; 1.0337x over previous
import functools

import jax
import jax.numpy as jnp
from jax import lax
from jax.experimental import pallas as pl
from jax.experimental.pallas import tpu as pltpu
from jax.experimental.pallas import tpu_sc as plsc

ROWS = 2048
COLS = 32768
TABLE = 256
LANES = 16
NWORKERS = 32
W = COLS // NWORKERS
RC = 32
NCHUNKS = ROWS // RC

_MESH = plsc.VectorSubcoreMesh(core_axis_name="c", subcore_axis_name="s")
_CPARAMS = pltpu.CompilerParams(needs_layout_passes=False,
                                use_tc_tiling_on_sc=True)


def _worker_col0():
    info = plsc.get_sparse_core_info()
    wid = lax.axis_index("s") * info.num_cores + lax.axis_index("c")
    return wid * W


def _p1_body(x_hbm, den_hbm, ds_hbm,
             xbuf0, xbuf1, denbuf, dsbuf, semx0, semx1):
    col0 = _worker_col0()
    xb = (xbuf0, xbuf1)
    sx = (semx0, semx1)

    def xslice(ch):
        return x_hbm.at[pl.ds(ch * RC, RC), pl.ds(col0, W)]

    pltpu.sync_copy(den_hbm, denbuf)

    def zero_cg(cg, c):
        dsbuf[pl.ds(cg * LANES, LANES)] = jnp.zeros((LANES,), jnp.float32)
        return c
    lax.fori_loop(0, W // LANES, zero_cg, 0)

    pltpu.async_copy(xslice(0), xb[0], sx[0])

    def p1_pair(p, c):
        for b in range(2):
            ch = p * 2 + b

            @pl.when(ch + 1 < NCHUNKS)
            def _():
                pltpu.async_copy(xslice(ch + 1), xb[1 - b], sx[1 - b])

            pltpu.make_async_copy(xslice(ch), xb[b], sx[b]).wait()
            xbuf = xb[b]

            def p1_cg(cg, cc):
                sl = pl.ds(cg * LANES, LANES)
                accs = [jnp.zeros((LANES,), jnp.float32) for _ in range(4)]
                for r in range(RC):
                    idx = xbuf[r, sl]
                    accs[r % 4] = accs[r % 4] + plsc.load_gather(denbuf, [idx])
                acc = (accs[0] + accs[1]) + (accs[2] + accs[3])
                dsbuf[sl] = dsbuf[sl] + acc
                return cc
            lax.fori_loop(0, W // LANES, p1_cg, 0)
        return c
    lax.fori_loop(0, NCHUNKS // 2, p1_pair, 0)

    pltpu.sync_copy(dsbuf, ds_hbm.at[pl.ds(col0, W)])


def _p2_body(x_hbm, num_hbm, recip_hbm, out_hbm,
             xbuf0, xbuf1, obuf0, obuf1, numbuf, recipbuf,
             semx0, semx1, semo0, semo1):
    col0 = _worker_col0()
    xb = (xbuf0, xbuf1)
    ob = (obuf0, obuf1)
    sx = (semx0, semx1)
    so = (semo0, semo1)

    def xslice(ch):
        return x_hbm.at[pl.ds(ch * RC, RC), pl.ds(col0, W)]

    def oslice(ch):
        return out_hbm.at[pl.ds(ch * (RC // 4), RC // 4), pl.ds(col0, W)]

    pltpu.sync_copy(num_hbm, numbuf)
    pltpu.sync_copy(recip_hbm.at[pl.ds(col0, W)], recipbuf)

    pltpu.async_copy(xslice(0), xb[0], sx[0])

    def p2_pair(p, c):
        for b in range(2):
            ch = p * 2 + b

            @pl.when(ch + 1 < NCHUNKS)
            def _():
                pltpu.async_copy(xslice(ch + 1), xb[1 - b], sx[1 - b])

            pltpu.make_async_copy(xslice(ch), xb[b], sx[b]).wait()

            @pl.when(ch >= 2)
            def _():
                pltpu.make_async_copy(ob[b], oslice(ch - 2), so[b]).wait()

            xbuf = xb[b]
            obuf = ob[b]

            def p2_cg(cg, cc):
                sl = pl.ds(cg * LANES, LANES)
                rec = recipbuf[sl]

                def p2_row4(r4, rr):
                    packed = jnp.zeros((LANES,), jnp.int32)
                    for k in range(4):
                        xv = xbuf[r4 * 4 + k, sl]
                        nv = plsc.load_gather(numbuf, [xv])
                        yi = jnp.minimum((nv * rec).astype(jnp.int32), 255)
                        packed = packed | (yi << (8 * k))
                    obuf[r4, sl] = packed
                    return rr
                lax.fori_loop(0, RC // 4, p2_row4, 0)
                return cc
            lax.fori_loop(0, W // LANES, p2_cg, 0)
            pltpu.async_copy(obuf, oslice(ch), so[b])
        return c
    lax.fori_loop(0, NCHUNKS // 2, p2_pair, 0)

    pltpu.make_async_copy(ob[0], oslice(NCHUNKS - 2), so[0]).wait()
    pltpu.make_async_copy(ob[1], oslice(NCHUNKS - 1), so[1]).wait()


_sc_pass1 = functools.partial(
    pl.kernel,
    out_type=jax.ShapeDtypeStruct((COLS,), jnp.float32),
    mesh=_MESH, compiler_params=_CPARAMS,
    scratch_types=[
        pltpu.VMEM((RC, W), jnp.int32),
        pltpu.VMEM((RC, W), jnp.int32),
        pltpu.VMEM((TABLE,), jnp.float32),
        pltpu.VMEM((W,), jnp.float32),
        pltpu.SemaphoreType.DMA,
        pltpu.SemaphoreType.DMA,
    ],
)(_p1_body)

_sc_pass2 = functools.partial(
    pl.kernel,
    out_type=jax.ShapeDtypeStruct((ROWS // 4, COLS), jnp.int32),
    mesh=_MESH, compiler_params=_CPARAMS,
    scratch_types=[
        pltpu.VMEM((RC, W), jnp.int32),
        pltpu.VMEM((RC, W), jnp.int32),
        pltpu.VMEM((RC // 4, W), jnp.int32),
        pltpu.VMEM((RC // 4, W), jnp.int32),
        pltpu.VMEM((TABLE,), jnp.float32),
        pltpu.VMEM((W,), jnp.float32),
        pltpu.SemaphoreType.DMA,
        pltpu.SemaphoreType.DMA,
        pltpu.SemaphoreType.DMA,
        pltpu.SemaphoreType.DMA,
    ],
)(_p2_body)


def kernel(x, denominator_table, numerator_table):
    ds = _sc_pass1(x, denominator_table)
    recip = jnp.float32(1.0) / ds
    packed = _sc_pass2(x, numerator_table, recip)
    rows = jnp.repeat(packed, 4, axis=0)
    sh = ((jnp.arange(ROWS, dtype=jnp.int32) % 4) * 8)[:, None]
    return ((rows >> sh) & 255).astype(jnp.uint8)
